# baseline (speedup 1.0000x reference)
import functools

import jax
import jax.numpy as jnp
from jax import lax
from jax.experimental import pallas as pl
from jax.experimental.pallas import tpu as pltpu
from jax.experimental.pallas import tpu_sc as plsc

N_NODES = 50000
N_PAD = 51200
N_EDGES = 3200000
LANE = 128
KROW = 8
E_PAD = 3276800
E_EXTRA = 2 * 1024
E_ROWS = E_PAD // LANE
NC, NS = 2, 16
CH = 1024


def _mesh():
    return plsc.VectorSubcoreMesh(core_axis_name="c", subcore_axis_name="s",
                                  num_cores=NC, num_subcores=NS)


def _make_prop(n_pad, e_pad, feat_split):
    nworkers = NS if feat_split else NC * NS
    edges_per_tile = e_pad // nworkers
    nbody = edges_per_tile // (4 * CH)
    rpt = n_pad // NS

    @functools.partial(
        pl.kernel,
        out_type=(
            jax.ShapeDtypeStruct((n_pad, 16), jnp.float32),
            jax.ShapeDtypeStruct((n_pad, 16), jnp.float32),
        ),
        mesh=_mesh(),
        compiler_params=pltpu.CompilerParams(use_tc_tiling_on_sc=False),
        scratch_types=[
            pltpu.VMEM_SHARED((n_pad, 16), jnp.float32),
        ] + [pltpu.VMEM((CH,), jnp.int32)] * 8
          + [pltpu.VMEM((CH, 16), jnp.float32)] * 4
          + [pltpu.SemaphoreType.DMA] * 12,
    )
    def prop_kernel(tbl_hbm, srca_hbm, srcb_hbm, dst_hbm, zeros_hbm, o0, o1,
                    acc_sh, sxa, sxb, sxc, sxd, dxa, dxb, dxc, dxd,
                    rowsa, rowsb, rowsc, rowsd,
                    mia, mib, mic, mid, sga, sgb, sgc, sgd,
                    ssa, ssb, ssc, ssd):
        cid = lax.axis_index("c")
        sid = lax.axis_index("s")
        nd = pl.ds(sid * rpt, rpt)
        pltpu.sync_copy(zeros_hbm.at[nd], acc_sh.at[nd])
        plsc.subcore_barrier()

        if feat_split:
            base = sid * edges_per_tile
        else:
            base = (cid * NS + sid) * edges_per_tile

        def load_idx(eo, sx, dx, mi):
            s_slice = pl.ds(eo, CH)
            if feat_split:
                @pl.when(cid == 0)
                def _():
                    pltpu.async_copy(srca_hbm.at[s_slice], sx, mi)

                @pl.when(cid == 1)
                def _():
                    pltpu.async_copy(srcb_hbm.at[s_slice], sx, mi)
                d1 = pltpu.make_async_copy(srca_hbm.at[s_slice], sx, mi)
            else:
                d1 = pltpu.async_copy(srca_hbm.at[s_slice], sx, mi)
            d2 = pltpu.async_copy(dst_hbm.at[s_slice], dx, mi)
            return (d1, d2)

        def drain_idx(descs):
            for d in descs:
                d.wait()

        pa0 = load_idx(base + 0 * CH, sxa, dxa, mia)
        pb0 = load_idx(base + 1 * CH, sxb, dxb, mib)

        def body(t, carry):
            e0 = base + t * 4 * CH
            pc = load_idx(e0 + 2 * CH, sxc, dxc, mic)
            pd = load_idx(e0 + 3 * CH, sxd, dxd, mid)
            drain_idx(load_idx_descr(e0 + 0 * CH, sxa, dxa, mia))
            drain_idx(load_idx_descr(e0 + 1 * CH, sxb, dxb, mib))
            ga = pltpu.async_copy(tbl_hbm.at[sxa], rowsa, sga)
            gb = pltpu.async_copy(tbl_hbm.at[sxb], rowsb, sgb)
            ga.wait()
            sa = pltpu.async_copy(rowsa, acc_sh.at[dxa], ssa, add=True)
            gb.wait()
            sb = pltpu.async_copy(rowsb, acc_sh.at[dxb], ssb, add=True)
            load_idx(e0 + 4 * CH, sxa, dxa, mia)
            load_idx(e0 + 5 * CH, sxb, dxb, mib)
            drain_idx(pc)
            drain_idx(pd)
            gc_ = pltpu.async_copy(tbl_hbm.at[sxc], rowsc, sgc)
            gd_ = pltpu.async_copy(tbl_hbm.at[sxd], rowsd, sgd)
            gc_.wait()
            sc_ = pltpu.async_copy(rowsc, acc_sh.at[dxc], ssc, add=True)
            gd_.wait()
            sd_ = pltpu.async_copy(rowsd, acc_sh.at[dxd], ssd, add=True)
            sa.wait()
            sb.wait()
            sc_.wait()
            sd_.wait()
            return carry

        def load_idx_descr(eo, sx, dx, mi):
            s_slice = pl.ds(eo, CH)
            d1 = pltpu.make_async_copy(srca_hbm.at[s_slice], sx, mi)
            d2 = pltpu.make_async_copy(dst_hbm.at[s_slice], dx, mi)
            return (d1, d2)

        lax.fori_loop(0, nbody, body, 0)
        drain_idx(load_idx_descr(base, sxa, dxa, mia))
        drain_idx(load_idx_descr(base, sxb, dxb, mib))
        plsc.subcore_barrier()

        @pl.when(cid == 0)
        def _():
            pltpu.sync_copy(acc_sh.at[nd], o0.at[nd])

        @pl.when(cid == 1)
        def _():
            pltpu.sync_copy(acc_sh.at[nd], o1.at[nd])

    return prop_kernel


def _make_deg(n_pad, e_pad):
    edges_per_tile = e_pad // (NC * NS)
    nbody = edges_per_tile // (2 * CH)
    rpt = n_pad // NS

    @functools.partial(
        pl.kernel,
        out_type=(
            jax.ShapeDtypeStruct((n_pad, 16), jnp.float32),
            jax.ShapeDtypeStruct((n_pad, 16), jnp.float32),
        ),
        mesh=_mesh(),
        compiler_params=pltpu.CompilerParams(use_tc_tiling_on_sc=False),
        scratch_types=[
            pltpu.VMEM_SHARED((n_pad, 16), jnp.float32),
            pltpu.VMEM((CH,), jnp.int32),
            pltpu.VMEM((CH,), jnp.int32),
            pltpu.VMEM((CH, 16), jnp.float32),
            pltpu.SemaphoreType.DMA,
            pltpu.SemaphoreType.DMA,
        ],
    )
    def deg_kernel(dst_hbm, zeros_hbm, ones_hbm, o0, o1,
                   acc_sh, dxa, dxb, ones_v, ssa, ssb):
        cid = lax.axis_index("c")
        sid = lax.axis_index("s")
        nd = pl.ds(sid * rpt, rpt)
        pltpu.sync_copy(zeros_hbm.at[nd], acc_sh.at[nd])
        pltpu.sync_copy(ones_hbm, ones_v)
        plsc.subcore_barrier()
        base = (cid * NS + sid) * edges_per_tile

        def body(t, carry):
            e0 = base + t * 2 * CH
            pltpu.sync_copy(dst_hbm.at[pl.ds(e0, CH)], dxa)
            sda = pltpu.async_copy(ones_v, acc_sh.at[dxa], ssa, add=True)
            pltpu.sync_copy(dst_hbm.at[pl.ds(e0 + CH, CH)], dxb)
            sdb = pltpu.async_copy(ones_v, acc_sh.at[dxb], ssb, add=True)
            sda.wait()
            sdb.wait()
            return carry

        lax.fori_loop(0, nbody, body, 0)
        plsc.subcore_barrier()

        @pl.when(cid == 0)
        def _():
            pltpu.sync_copy(acc_sh.at[nd], o0.at[nd])

        @pl.when(cid == 1)
        def _():
            pltpu.sync_copy(acc_sh.at[nd], o1.at[nd])

    return deg_kernel


def _dis(cnt0_ref, cnt1_ref):
    return lax.rsqrt(cnt0_ref[:, :1] + cnt1_ref[:, :1] + 1.0)


def _stage_a(cnt0_ref, cnt1_ref, x_ref, w_ref, o_ref):
    dis = _dis(cnt0_ref, cnt1_ref)
    o_ref[...] = dis * jnp.dot(x_ref[...], w_ref[...],
                               preferred_element_type=jnp.float32)


def _stage_b(s0, s1, hp, cnt0, cnt1, b, w, o):
    dis = _dis(cnt0, cnt1)
    act = jax.nn.relu(dis * (s0[...] + s1[...] + hp[...]) + b[...])
    o[...] = dis * jnp.dot(act, w[...], preferred_element_type=jnp.float32)


def _stage_b1_body(s0, s1, hp, cnt0, cnt1, b, w, oa, ob):
    dis = _dis(cnt0, cnt1)
    act = jax.nn.relu(dis * (s0[...] + s1[...] + hp[...]) + b[...])
    h2 = dis * jnp.dot(act, w[...], preferred_element_type=jnp.float32)
    oa[...] = h2[:, :16]
    ob[...] = h2[:, 16:]


def _tc_call_b1(ins, n_rows, r):
    grid = (n_rows // r,)
    widths = (16, 16, 16, 16, 16, None, None)
    in_specs = []
    for a, w in zip(ins, widths):
        if w is None:
            in_specs.append(pl.BlockSpec(a.shape, lambda i, nd=a.ndim: (0,) * nd))
        else:
            in_specs.append(pl.BlockSpec((r, w), lambda i: (i, 0)))
    out_shape = tuple(jax.ShapeDtypeStruct((n_rows, 16), jnp.float32) for _ in range(2))
    out_specs = tuple(pl.BlockSpec((r, 16), lambda i: (i, 0)) for _ in range(2))
    return pl.pallas_call(_stage_b1_body, grid=grid, in_specs=in_specs,
                          out_specs=out_specs, out_shape=out_shape)(*ins)


def _stage_b2(s2a, s2b, h2a, h2b, cnt0, cnt1, b, w, o):
    dis = _dis(cnt0, cnt1)
    s2 = jnp.concatenate([s2a[...], s2b[...]], axis=1)
    hp = jnp.concatenate([h2a[...], h2b[...]], axis=1)
    act = jax.nn.relu(dis * (s2 + hp) + b[...])
    o[...] = dis * jnp.dot(act, w[...], preferred_element_type=jnp.float32)


def _stage_c(s0, s1, hp, cnt0, cnt1, b, o):
    dis = _dis(cnt0, cnt1)
    act = jax.nn.relu(dis * (s0[...] + s1[...] + hp[...]) + b[...])
    o[...] = act[:, :11]


def _tc_call(body, out_width, ins, widths, n_rows, r):
    grid = (n_rows // r,)
    in_specs = []
    for a, w in zip(ins, widths):
        if w is None:
            in_specs.append(pl.BlockSpec(a.shape, lambda i, nd=a.ndim: (0,) * nd))
        else:
            in_specs.append(pl.BlockSpec((r, w), lambda i: (i, 0)))
    out_shape = jax.ShapeDtypeStruct((n_rows, out_width), jnp.float32)
    out_specs = pl.BlockSpec((r, out_width), lambda i: (i, 0))
    return pl.pallas_call(body, grid=grid, in_specs=in_specs,
                          out_specs=out_specs, out_shape=out_shape)(*ins)


def kernel(x, edge_index, W1, b1, W2, b2, W3, b3, W4, b4):
    f32 = jnp.float32
    src = edge_index[0].astype(jnp.int32)
    dst = edge_index[1].astype(jnp.int32)
    pad = jnp.full((E_PAD + E_EXTRA - N_EDGES,), N_NODES, jnp.int32)
    src1 = jnp.concatenate([src, pad])
    dst1 = jnp.concatenate([dst, pad])

    xpad = jnp.zeros((N_PAD, 16), f32).at[:N_NODES, :11].set(x)
    w1p = jnp.zeros((16, 16), f32).at[:11, :].set(W1)
    w4p = jnp.zeros((16, 16), f32).at[:, :11].set(W4)
    b1r = b1.reshape(1, 16)
    b2r = b2.reshape(1, 32)
    b3r = b3.reshape(1, 16)
    b4r = jnp.zeros((1, 16), f32).at[0, :11].set(b4)
    zeros16 = jnp.zeros((N_PAD, 16), f32)
    zeros32 = jnp.zeros((N_PAD, 32), f32)
    zeros32 = jnp.zeros((N_PAD, 32), f32)
    ones_hbm = jnp.ones((CH, 16), f32)

    deg_k = _make_deg(N_PAD, E_PAD)
    prop_e = _make_prop(N_PAD, E_PAD, feat_split=False)
    prop_f = _make_prop(N_PAD, E_PAD, feat_split=True)

    srcoff1 = src1 + N_PAD

    cnt0, cnt1 = deg_k(dst1, zeros16, ones_hbm)

    R = 1600
    hp1 = _tc_call(_stage_a, 16, (cnt0, cnt1, xpad, w1p),
                   (16, 16, 16, None), N_PAD, R)
    s10, s11 = prop_e(hp1, src1, src1, dst1, zeros16)
    hp2a, hp2b = _tc_call_b1((s10, s11, hp1, cnt0, cnt1, b1r, W2), N_PAD, R)
    tbl2 = jnp.concatenate([hp2a, hp2b], axis=0)
    s2a, s2b = prop_f(tbl2, src1, srcoff1, dst1, zeros16)
    hp3 = _tc_call(_stage_b2, 16, (s2a, s2b, hp2a, hp2b, cnt0, cnt1, b2r, W3),
                   (16, 16, 16, 16, 16, 16, None, None), N_PAD, R)
    s30, s31 = prop_e(hp3, src1, src1, dst1, zeros16)
    hp4 = _tc_call(_stage_b, 16, (s30, s31, hp3, cnt0, cnt1, b3r, w4p),
                   (16, 16, 16, 16, 16, None, None), N_PAD, R)
    s40, s41 = prop_e(hp4, src1, src1, dst1, zeros16)
    out = _tc_call(_stage_c, 11, (s40, s41, hp4, cnt0, cnt1, b4r),
                   (16, 16, 16, 16, 16, None), N_NODES, 2000)
    return out


# speedup vs baseline: 1.6313x; 1.6313x over previous
import functools

import jax
import jax.numpy as jnp
from jax import lax
from jax.experimental import pallas as pl
from jax.experimental.pallas import tpu as pltpu
from jax.experimental.pallas import tpu_sc as plsc

N_NODES = 50000
N_PAD = 51200
N_EDGES = 3200000
LANE = 128
KROW = 8
E_PAD = 3211264
E_ROWS = E_PAD // LANE
NC, NS = 2, 16
CH = 1024


def _mesh():
    return plsc.VectorSubcoreMesh(core_axis_name="c", subcore_axis_name="s",
                                  num_cores=NC, num_subcores=NS)


def _make_prop(n_pad, e_pad, feat_split):
    nworkers = NS if feat_split else NC * NS
    edges_per_tile = e_pad // nworkers
    nbody = edges_per_tile // (2 * CH)
    rpt = n_pad // NS

    @functools.partial(
        pl.kernel,
        out_type=(
            jax.ShapeDtypeStruct((n_pad, 16), jnp.float32),
            jax.ShapeDtypeStruct((n_pad, 16), jnp.float32),
        ),
        mesh=_mesh(),
        compiler_params=pltpu.CompilerParams(use_tc_tiling_on_sc=False),
        scratch_types=[
            pltpu.VMEM_SHARED((n_pad, 16), jnp.float32),
            pltpu.VMEM((CH,), jnp.int32),
            pltpu.VMEM((CH,), jnp.int32),
            pltpu.VMEM((CH,), jnp.int32),
            pltpu.VMEM((CH,), jnp.int32),
            pltpu.VMEM((CH, 16), jnp.float32),
            pltpu.VMEM((CH, 16), jnp.float32),
            pltpu.SemaphoreType.DMA,
            pltpu.SemaphoreType.DMA,
            pltpu.SemaphoreType.DMA,
            pltpu.SemaphoreType.DMA,
        ],
    )
    def prop_kernel(tbl_hbm, srca_hbm, srcb_hbm, dst_hbm, zeros_hbm, o0, o1,
                    acc_sh, sxa, sxb, dxa, dxb, rowsa, rowsb,
                    sga, sgb, ssa, ssb):
        cid = lax.axis_index("c")
        sid = lax.axis_index("s")
        nd = pl.ds(sid * rpt, rpt)
        pltpu.sync_copy(zeros_hbm.at[nd], acc_sh.at[nd])
        plsc.subcore_barrier()

        if feat_split:
            base = sid * edges_per_tile
        else:
            base = (cid * NS + sid) * edges_per_tile

        def body(t, carry):
            e0 = base + t * 2 * CH

            def bank(eo, sx, dx, rows, sg):
                @pl.when(cid == 0)
                def _():
                    pltpu.sync_copy(srca_hbm.at[pl.ds(eo, CH)], sx)

                @pl.when(cid == 1)
                def _():
                    pltpu.sync_copy(srcb_hbm.at[pl.ds(eo, CH)], sx)
                pltpu.sync_copy(dst_hbm.at[pl.ds(eo, CH)], dx)
                return pltpu.async_copy(tbl_hbm.at[sx], rows, sg)

            gda = bank(e0, sxa, dxa, rowsa, sga)
            gdb = bank(e0 + CH, sxb, dxb, rowsb, sgb)
            gda.wait()
            sda = pltpu.async_copy(rowsa, acc_sh.at[dxa], ssa, add=True)
            gdb.wait()
            sdb = pltpu.async_copy(rowsb, acc_sh.at[dxb], ssb, add=True)
            sda.wait()
            sdb.wait()
            return carry

        lax.fori_loop(0, nbody, body, 0)
        plsc.subcore_barrier()

        @pl.when(cid == 0)
        def _():
            pltpu.sync_copy(acc_sh.at[nd], o0.at[nd])

        @pl.when(cid == 1)
        def _():
            pltpu.sync_copy(acc_sh.at[nd], o1.at[nd])

    return prop_kernel


def _make_deg(n_pad, e_pad):
    edges_per_tile = e_pad // (NC * NS)
    nbody = edges_per_tile // (2 * CH)
    rpt = n_pad // NS

    @functools.partial(
        pl.kernel,
        out_type=(
            jax.ShapeDtypeStruct((n_pad, 16), jnp.float32),
            jax.ShapeDtypeStruct((n_pad, 16), jnp.float32),
        ),
        mesh=_mesh(),
        compiler_params=pltpu.CompilerParams(use_tc_tiling_on_sc=False),
        scratch_types=[
            pltpu.VMEM_SHARED((n_pad, 16), jnp.float32),
            pltpu.VMEM((CH,), jnp.int32),
            pltpu.VMEM((CH,), jnp.int32),
            pltpu.VMEM((CH, 16), jnp.float32),
            pltpu.SemaphoreType.DMA,
            pltpu.SemaphoreType.DMA,
        ],
    )
    def deg_kernel(dst_hbm, zeros_hbm, ones_hbm, o0, o1,
                   acc_sh, dxa, dxb, ones_v, ssa, ssb):
        cid = lax.axis_index("c")
        sid = lax.axis_index("s")
        nd = pl.ds(sid * rpt, rpt)
        pltpu.sync_copy(zeros_hbm.at[nd], acc_sh.at[nd])
        pltpu.sync_copy(ones_hbm, ones_v)
        plsc.subcore_barrier()
        base = (cid * NS + sid) * edges_per_tile

        def body(t, carry):
            e0 = base + t * 2 * CH
            pltpu.sync_copy(dst_hbm.at[pl.ds(e0, CH)], dxa)
            sda = pltpu.async_copy(ones_v, acc_sh.at[dxa], ssa, add=True)
            pltpu.sync_copy(dst_hbm.at[pl.ds(e0 + CH, CH)], dxb)
            sdb = pltpu.async_copy(ones_v, acc_sh.at[dxb], ssb, add=True)
            sda.wait()
            sdb.wait()
            return carry

        lax.fori_loop(0, nbody, body, 0)
        plsc.subcore_barrier()

        @pl.when(cid == 0)
        def _():
            pltpu.sync_copy(acc_sh.at[nd], o0.at[nd])

        @pl.when(cid == 1)
        def _():
            pltpu.sync_copy(acc_sh.at[nd], o1.at[nd])

    return deg_kernel


def _dis(cnt0_ref, cnt1_ref):
    return lax.rsqrt(cnt0_ref[:, :1] + cnt1_ref[:, :1] + 1.0)


def _stage_a(cnt0_ref, cnt1_ref, x_ref, w_ref, o_ref):
    dis = _dis(cnt0_ref, cnt1_ref)
    o_ref[...] = dis * jnp.dot(x_ref[...], w_ref[...],
                               preferred_element_type=jnp.float32)


def _stage_b(s0, s1, hp, cnt0, cnt1, b, w, o):
    dis = _dis(cnt0, cnt1)
    act = jax.nn.relu(dis * (s0[...] + s1[...] + hp[...]) + b[...])
    o[...] = dis * jnp.dot(act, w[...], preferred_element_type=jnp.float32)


def _stage_b1_body(s0, s1, hp, cnt0, cnt1, b, w, oa, ob):
    dis = _dis(cnt0, cnt1)
    act = jax.nn.relu(dis * (s0[...] + s1[...] + hp[...]) + b[...])
    h2 = dis * jnp.dot(act, w[...], preferred_element_type=jnp.float32)
    oa[...] = h2[:, :16]
    ob[...] = h2[:, 16:]


def _tc_call_b1(ins, n_rows, r):
    grid = (n_rows // r,)
    widths = (16, 16, 16, 16, 16, None, None)
    in_specs = []
    for a, w in zip(ins, widths):
        if w is None:
            in_specs.append(pl.BlockSpec(a.shape, lambda i, nd=a.ndim: (0,) * nd))
        else:
            in_specs.append(pl.BlockSpec((r, w), lambda i: (i, 0)))
    out_shape = tuple(jax.ShapeDtypeStruct((n_rows, 16), jnp.float32) for _ in range(2))
    out_specs = tuple(pl.BlockSpec((r, 16), lambda i: (i, 0)) for _ in range(2))
    return pl.pallas_call(_stage_b1_body, grid=grid, in_specs=in_specs,
                          out_specs=out_specs, out_shape=out_shape)(*ins)


def _stage_b2(s2a, s2b, h2a, h2b, cnt0, cnt1, b, w, o):
    dis = _dis(cnt0, cnt1)
    s2 = jnp.concatenate([s2a[...], s2b[...]], axis=1)
    hp = jnp.concatenate([h2a[...], h2b[...]], axis=1)
    act = jax.nn.relu(dis * (s2 + hp) + b[...])
    o[...] = dis * jnp.dot(act, w[...], preferred_element_type=jnp.float32)


def _stage_c(s0, s1, hp, cnt0, cnt1, b, o):
    dis = _dis(cnt0, cnt1)
    act = jax.nn.relu(dis * (s0[...] + s1[...] + hp[...]) + b[...])
    o[...] = act[:, :11]


def _tc_call(body, out_width, ins, widths, n_rows, r):
    grid = (n_rows // r,)
    in_specs = []
    for a, w in zip(ins, widths):
        if w is None:
            in_specs.append(pl.BlockSpec(a.shape, lambda i, nd=a.ndim: (0,) * nd))
        else:
            in_specs.append(pl.BlockSpec((r, w), lambda i: (i, 0)))
    out_shape = jax.ShapeDtypeStruct((n_rows, out_width), jnp.float32)
    out_specs = pl.BlockSpec((r, out_width), lambda i: (i, 0))
    return pl.pallas_call(body, grid=grid, in_specs=in_specs,
                          out_specs=out_specs, out_shape=out_shape)(*ins)


def kernel(x, edge_index, W1, b1, W2, b2, W3, b3, W4, b4):
    f32 = jnp.float32
    src = edge_index[0].astype(jnp.int32)
    dst = edge_index[1].astype(jnp.int32)
    pad = jnp.full((E_PAD - N_EDGES,), N_NODES, jnp.int32)
    src1 = jnp.concatenate([src, pad])
    dst1 = jnp.concatenate([dst, pad])

    xpad = jnp.zeros((N_PAD, 16), f32).at[:N_NODES, :11].set(x)
    w1p = jnp.zeros((16, 16), f32).at[:11, :].set(W1)
    w4p = jnp.zeros((16, 16), f32).at[:, :11].set(W4)
    b1r = b1.reshape(1, 16)
    b2r = b2.reshape(1, 32)
    b3r = b3.reshape(1, 16)
    b4r = jnp.zeros((1, 16), f32).at[0, :11].set(b4)
    zeros16 = jnp.zeros((N_PAD, 16), f32)
    zeros32 = jnp.zeros((N_PAD, 32), f32)
    zeros32 = jnp.zeros((N_PAD, 32), f32)
    ones_hbm = jnp.ones((CH, 16), f32)

    deg_k = _make_deg(N_PAD, E_PAD)
    prop_e = _make_prop(N_PAD, E_PAD, feat_split=False)
    prop_f = _make_prop(N_PAD, E_PAD, feat_split=True)

    srcoff1 = src1 + N_PAD

    cnt0, cnt1 = deg_k(dst1, zeros16, ones_hbm)

    R = 1600
    hp1 = _tc_call(_stage_a, 16, (cnt0, cnt1, xpad, w1p),
                   (16, 16, 16, None), N_PAD, R)
    s10, s11 = prop_e(hp1, src1, src1, dst1, zeros16)
    hp2a, hp2b = _tc_call_b1((s10, s11, hp1, cnt0, cnt1, b1r, W2), N_PAD, R)
    tbl2 = jnp.concatenate([hp2a, hp2b], axis=0)
    s2a, s2b = prop_f(tbl2, src1, srcoff1, dst1, zeros16)
    hp3 = _tc_call(_stage_b2, 16, (s2a, s2b, hp2a, hp2b, cnt0, cnt1, b2r, W3),
                   (16, 16, 16, 16, 16, 16, None, None), N_PAD, R)
    s30, s31 = prop_e(hp3, src1, src1, dst1, zeros16)
    hp4 = _tc_call(_stage_b, 16, (s30, s31, hp3, cnt0, cnt1, b3r, w4p),
                   (16, 16, 16, 16, 16, None, None), N_PAD, R)
    s40, s41 = prop_e(hp4, src1, src1, dst1, zeros16)
    out = _tc_call(_stage_c, 11, (s40, s41, hp4, cnt0, cnt1, b4r),
                   (16, 16, 16, 16, 16, None), N_NODES, 2000)
    return out
